# baseline (speedup 1.0000x reference)
import functools

import jax
import jax.numpy as jnp
from jax import lax
from jax.experimental import pallas as pl
from jax.experimental.pallas import tpu as pltpu
from jax.experimental.pallas import tpu_sc as plsc

NUM_EMB = 1000000
EMB_DIM = 32
BATCH = 16384

_info = plsc.get_sparse_core_info()
_NC, _NS = _info.num_cores, _info.num_subcores
_NW = _NC * _NS
_BPW = BATCH // _NW
_CHUNK = 128
_NCH = _BPW // _CHUNK
_TROWS = NUM_EMB * EMB_DIM // 128

_mesh = plsc.VectorSubcoreMesh(core_axis_name="c", subcore_axis_name="s")


@functools.partial(
    pl.kernel,
    mesh=_mesh,
    out_type=jax.ShapeDtypeStruct((BATCH * 2 * EMB_DIM // 128, 128),
                                  jnp.float32),
    scratch_types=[
        pltpu.VMEM((8, 128), jnp.int32),
        pltpu.VMEM((8, 128), jnp.int32),
        pltpu.VMEM((_NCH, _CHUNK), jnp.int32),
        pltpu.VMEM((_NCH, _CHUNK), jnp.int32),
        pltpu.VMEM((2, _CHUNK, 128), jnp.float32),
        pltpu.VMEM((2, _CHUNK, 128), jnp.float32),
        pltpu.VMEM((_BPW // 2, 128), jnp.float32),
        pltpu.SemaphoreType.DMA,
        pltpu.SemaphoreType.DMA,
    ],
)
def _gather_concat(subj_ids, obj_ids, subj128, obj128, out,
                   sraw, oraw, sidxv, oidxv, sbuf, obuf, comb,
                   sem0, sem1):
    wid = lax.axis_index("s") * _NC + lax.axis_index("c")
    sems = (sem0, sem1)

    h = (wid & 1) * _NCH
    pltpu.sync_copy(subj_ids.at[pl.ds((wid >> 1) * 8, 8)], sraw)
    pltpu.sync_copy(obj_ids.at[pl.ds((wid >> 1) * 8, 8)], oraw)

    for r in range(_NCH):
        for g in range(_CHUNK // 16):
            sl = pl.ds(16 * g, 16)
            sidxv[r, sl] = lax.shift_right_logical(sraw[h + r, sl], 2)
            oidxv[r, sl] = lax.shift_right_logical(oraw[h + r, sl], 2)

    def fire(j):
        b = j & 1
        return (
            pltpu.async_copy(subj128.at[sidxv.at[j]], sbuf.at[b], sems[b]),
            pltpu.async_copy(obj128.at[oidxv.at[j]], obuf.at[b], sems[b]),
        )

    def extract(j):
        b = j & 1

        def body(g, _):
            sq16 = lax.bitwise_and(sraw[h + j, pl.ds(g * 16, 16)], 3) * 32
            oq16 = lax.bitwise_and(oraw[h + j, pl.ds(g * 16, 16)], 3) * 32
            for i in range(16):
                sq, oq = sq16[i], oq16[i]
                k = g * 16 + i
                crow = j * (_CHUNK // 2) + g * 8 + (i >> 1)
                cb = (i & 1) * 64
                comb[crow, pl.ds(cb, 16)] = sbuf[b, k, pl.ds(sq, 16)]
                comb[crow, pl.ds(cb + 16, 16)] = \
                    sbuf[b, k, pl.ds(sq + 16, 16)]
                comb[crow, pl.ds(cb + 32, 16)] = obuf[b, k, pl.ds(oq, 16)]
                comb[crow, pl.ds(cb + 48, 16)] = \
                    obuf[b, k, pl.ds(oq + 16, 16)]
            return 0

        lax.fori_loop(0, _CHUNK // 16, body, 0)

    pending = fire(0)
    for j in range(_NCH):
        nxt = fire(j + 1) if j + 1 < _NCH else ()
        for c in pending:
            c.wait()
        extract(j)
        pending = nxt

    pltpu.sync_copy(comb, out.at[pl.ds(wid * (_BPW // 2), _BPW // 2)])


def kernel(entity_types, subj_table, obj_table):
    subj_ids = entity_types[:, 0].reshape(BATCH // 128, 128)
    obj_ids = entity_types[:, 1].reshape(BATCH // 128, 128)
    s128 = subj_table.reshape(_TROWS, 128)
    o128 = obj_table.reshape(_TROWS, 128)
    out = _gather_concat(subj_ids, obj_ids, s128, o128)
    return out.reshape(BATCH, 2 * EMB_DIM)


# speedup vs baseline: 1.5152x; 1.5152x over previous
import functools

import jax
import jax.numpy as jnp
from jax import lax
from jax.experimental import pallas as pl
from jax.experimental.pallas import tpu as pltpu
from jax.experimental.pallas import tpu_sc as plsc

NUM_EMB = 1000000
EMB_DIM = 32
BATCH = 16384

_info = plsc.get_sparse_core_info()
_NC, _NS = _info.num_cores, _info.num_subcores
_NW = _NC * _NS
_BPW = BATCH // _NW
_NG = _BPW // 16

_mesh = plsc.VectorSubcoreMesh(core_axis_name="c", subcore_axis_name="s")


@functools.partial(
    pl.kernel,
    mesh=_mesh,
    out_type=jax.ShapeDtypeStruct((BATCH, 2 * EMB_DIM), jnp.float32),
    scratch_types=[
        pltpu.VMEM((8, 128), jnp.int32),
        pltpu.VMEM((8, 128), jnp.int32),
        pltpu.VMEM((_BPW, 2 * EMB_DIM), jnp.float32),
        pltpu.SemaphoreType.DMA,
    ],
)
def _gather_concat(subj_ids, obj_ids, subj_tbl, obj_tbl, out,
                   sraw, oraw, comb, sem):
    wid = lax.axis_index("s") * _NC + lax.axis_index("c")

    h = (wid & 1) * 4
    pltpu.sync_copy(subj_ids.at[pl.ds((wid >> 1) * 8, 8)], sraw)
    pltpu.sync_copy(obj_ids.at[pl.ds((wid >> 1) * 8, 8)], oraw)

    def fire(g, _):
        r = h + lax.shift_right_logical(g, 3)
        c = lax.bitwise_and(g, 7) * 16
        sid16 = sraw[r, pl.ds(c, 16)]
        oid16 = oraw[r, pl.ds(c, 16)]
        for i in range(16):
            k = g * 16 + i
            pltpu.async_copy(subj_tbl.at[sid16[i]],
                             comb.at[k, pl.ds(0, EMB_DIM)], sem)
            pltpu.async_copy(obj_tbl.at[oid16[i]],
                             comb.at[k, pl.ds(EMB_DIM, EMB_DIM)], sem)
        return 0

    lax.fori_loop(0, _NG, fire, 0)

    def drain(g, _):
        for _i in range(32):
            pltpu.make_async_copy(subj_tbl.at[0],
                                  comb.at[0, pl.ds(0, EMB_DIM)], sem).wait()
        return 0

    lax.fori_loop(0, _NG, drain, 0)

    pltpu.sync_copy(comb, out.at[pl.ds(wid * _BPW, _BPW)])


def kernel(entity_types, subj_table, obj_table):
    subj_ids = entity_types[:, 0].reshape(BATCH // 128, 128)
    obj_ids = entity_types[:, 1].reshape(BATCH // 128, 128)
    return _gather_concat(subj_ids, obj_ids, subj_table, obj_table)
